# baseline (speedup 1.0000x reference)
import jax
import jax.numpy as jnp
from jax import lax
from jax.experimental import pallas as pl
from jax.experimental.pallas import tpu as pltpu
from jax.experimental.pallas import tpu_sc as plsc

N_NODES = 10000
HROW = 640
NPAD = HROW * 16
NC, NS = 2, 16
NW = NC * NS
IDXB = 128
NB = 84
PER_TILE = NB * IDXB
E_PAD = NW * PER_TILE
NBATCH = E_PAD // IDXB
NB0 = 24
NB1 = (NBATCH - NS * NB0) // NS
BM = 2000


_SC_PARAMS = pltpu.CompilerParams(needs_layout_passes=False,
                                  use_tc_tiling_on_sc=False)


def _deg_body(col_hbm, idx_hbm, zeros_hbm, deg_hbm, colbuf, hist_v, idxv, shist):
    c = lax.axis_index("c")
    s = lax.axis_index("s")
    tid = c * NS + s
    pltpu.sync_copy(zeros_hbm, hist_v)
    pltpu.sync_copy(idx_hbm, idxv)
    pltpu.sync_copy(col_hbm.at[tid], colbuf)

    @pl.when(s == 0)
    def _():
        pltpu.sync_copy(zeros_hbm, shist)

    plsc.subcore_barrier()

    def outer(j, carry):
        def inner(l, carry):
            idx = colbuf[j, pl.ds(l * 16, 16)]
            cnt, last = plsc.scan_count(idx)
            hi = lax.shift_right_logical(idx, 4)
            lo = lax.bitwise_and(idx, 15)
            plsc.addupdate_scatter(hist_v, [hi, lo], cnt.astype(jnp.float32),
                                   mask=last)
            return carry
        return lax.fori_loop(0, IDXB // 16, inner, carry)

    lax.fori_loop(0, NB, outer, 0)

    for j in range(HROW // IDXB):
        pltpu.sync_copy(hist_v.at[pl.ds(j * IDXB, IDXB)],
                        shist.at[idxv.at[j]], add=True)
    plsc.subcore_barrier()

    @pl.when(s == 0)
    def _():
        pltpu.sync_copy(shist, deg_hbm.at[pl.ds(c * HROW, HROW)])


_deg_call = pl.kernel(
    _deg_body,
    out_type=jax.ShapeDtypeStruct((NC * HROW, 16), jnp.float32),
    mesh=plsc.VectorSubcoreMesh(core_axis_name="c", subcore_axis_name="s"),
    compiler_params=_SC_PARAMS,
    scratch_types=[
        pltpu.VMEM((NB, IDXB), jnp.int32),
        pltpu.VMEM((HROW, 16), jnp.float32),
        pltpu.VMEM((HROW // IDXB, IDXB), jnp.int32),
        pltpu.VMEM_SHARED((HROW, 16), jnp.float32),
    ],
)


NRING = 4


def _mp_body(xs_hbm, row_hbm, col_hbm, zeros_hbm, acc_hbm,
             rbuf, cbuf, g0, g1, g2, g3, acc_s, gsem, ssem):
    c = lax.axis_index("c")
    s = lax.axis_index("s")
    bufs = (g0, g1, g2, g3)
    start = pl.multiple_of(jnp.where(c == 0, s * NB0, NS * NB0 + s * NB1), 8)
    nb = jnp.where(c == 0, NB0, NB1)
    pltpu.sync_copy(row_hbm.at[pl.ds(start, NB1)], rbuf)
    pltpu.sync_copy(col_hbm.at[pl.ds(start, NB1)], cbuf)
    rows_per_tile = NPAD // NS
    pltpu.sync_copy(zeros_hbm.at[pl.ds(s * rows_per_tile, rows_per_tile)],
                    acc_s.at[pl.ds(s * rows_per_tile, rows_per_tile)])
    plsc.subcore_barrier()

    def wait_scat(buf):
        pltpu.make_async_copy(buf, acc_s.at[cbuf.at[0]], ssem).wait()

    for b in range(NRING):
        pltpu.async_copy(xs_hbm.at[rbuf.at[b]], bufs[b], gsem)

    def body(q, carry):
        j = q * NRING
        for b in range(NRING):
            pltpu.make_async_copy(xs_hbm.at[rbuf.at[j + b]], bufs[b],
                                  gsem).wait()
            pltpu.async_copy(bufs[b], acc_s.at[cbuf.at[j + b]], ssem,
                             add=True)
        for b in range(NRING):
            jn = j + NRING + b

            @pl.when(jn < nb)
            def _():
                wait_scat(bufs[b])
                pltpu.async_copy(xs_hbm.at[rbuf.at[jn]], bufs[b], gsem)

        return carry

    lax.fori_loop(0, nb // NRING, body, 0)
    for b in range(NRING):
        wait_scat(bufs[b])
    plsc.subcore_barrier()
    pltpu.sync_copy(acc_s.at[pl.ds(s * rows_per_tile, rows_per_tile)],
                    acc_hbm.at[pl.ds(c * NPAD + s * rows_per_tile,
                                     rows_per_tile)])


def _make_mp_call(feat):
    return pl.kernel(
        _mp_body,
        out_type=jax.ShapeDtypeStruct((NC * NPAD, feat), jnp.float32),
        mesh=plsc.VectorSubcoreMesh(core_axis_name="c", subcore_axis_name="s"),
        compiler_params=_SC_PARAMS,
        scratch_types=[
            pltpu.VMEM((NB1, IDXB), jnp.int32),
            pltpu.VMEM((NB1, IDXB), jnp.int32),
            pltpu.VMEM((IDXB, feat), jnp.float32),
            pltpu.VMEM((IDXB, feat), jnp.float32),
            pltpu.VMEM((IDXB, feat), jnp.float32),
            pltpu.VMEM((IDXB, feat), jnp.float32),
            pltpu.VMEM_SHARED((NPAD, feat), jnp.float32),
            pltpu.SemaphoreType.DMA,
            pltpu.SemaphoreType.DMA,
        ],
    )


_mp_calls = {feat: _make_mp_call(feat) for feat in (64, 32, 16)}


def _dis_block(deg):
    d = deg[:, 0] + deg[:, 1]
    return jnp.where(d > 0, lax.rsqrt(d), 0.0)[:, None]


def _mm1_body(deg_ref, x_ref, w_ref, o_ref):
    dis = _dis_block(deg_ref[...])
    o_ref[...] = dis * jnp.dot(x_ref[...], w_ref[...],
                               preferred_element_type=jnp.float32)


def _mid_body(deg_ref, a0_ref, a1_ref, b_ref, w_ref, o_ref):
    dis = _dis_block(deg_ref[...])
    h = jnp.maximum(dis * (a0_ref[...] + a1_ref[...]) + b_ref[...], 0.0)
    o_ref[...] = dis * jnp.dot(h, w_ref[...],
                               preferred_element_type=jnp.float32)


def _fin_body(deg_ref, a0_ref, a1_ref, b_ref, o_ref):
    dis = _dis_block(deg_ref[...])
    o_ref[...] = dis * (a0_ref[...] + a1_ref[...]) + b_ref[...]


def _row_spec(feat):
    return pl.BlockSpec((BM, feat), lambda m: (m, 0))


_DEG_SPEC = pl.BlockSpec((BM, 2), lambda m: (m, 0))
_GRID = (N_NODES // BM,)


def _mm1(deg, x, w):
    fin, fout = w.shape
    return pl.pallas_call(
        _mm1_body,
        grid=_GRID,
        in_specs=[_DEG_SPEC, _row_spec(fin),
                  pl.BlockSpec((fin, fout), lambda m: (0, 0))],
        out_specs=_row_spec(fout),
        out_shape=jax.ShapeDtypeStruct((N_NODES, fout), jnp.float32),
    )(deg, x, w)


def _mid(deg, a0, a1, b, w):
    fin, fout = w.shape
    return pl.pallas_call(
        _mid_body,
        grid=_GRID,
        in_specs=[_DEG_SPEC, _row_spec(fin), _row_spec(fin),
                  pl.BlockSpec((1, fin), lambda m: (0, 0)),
                  pl.BlockSpec((fin, fout), lambda m: (0, 0))],
        out_specs=_row_spec(fout),
        out_shape=jax.ShapeDtypeStruct((N_NODES, fout), jnp.float32),
    )(deg, a0, a1, b, w)


def _fin(deg, a0, a1, b):
    feat = a0.shape[1]
    return pl.pallas_call(
        _fin_body,
        grid=_GRID,
        in_specs=[_DEG_SPEC, _row_spec(feat), _row_spec(feat),
                  pl.BlockSpec((1, feat), lambda m: (0, 0))],
        out_specs=_row_spec(feat),
        out_shape=jax.ShapeDtypeStruct((N_NODES, feat), jnp.float32),
    )(deg, a0, a1, b)


def kernel(x, edge_index, W1, b1, W2, b2, W3, b3):
    n = x.shape[0]
    loop = jnp.arange(n, dtype=jnp.int32)
    row = jnp.concatenate([edge_index[0], loop])
    col = jnp.concatenate([edge_index[1], loop])
    npadding = E_PAD - row.shape[0]
    row_f = jnp.concatenate([row, jnp.zeros((npadding,), jnp.int32)])
    col_f = jnp.concatenate([col, jnp.full((npadding,), n, jnp.int32)])
    col_p3 = col_f.reshape(NW, NB, IDXB)
    row_p = row_f.reshape(NBATCH, IDXB)
    col_p = col_f.reshape(NBATCH, IDXB)
    idx_rows = jnp.arange(HROW, dtype=jnp.int32).reshape(HROW // IDXB, IDXB)
    zeros_deg = jnp.zeros((HROW, 16), jnp.float32)

    deg2 = _deg_call(col_p3, idx_rows, zeros_deg)
    deg2 = deg2.reshape(NC, NPAD)[:, :n].T

    def agg(xs):
        feat = xs.shape[1]
        accf = _mp_calls[feat](xs, row_p, col_p,
                               jnp.zeros((NPAD, feat), jnp.float32))
        return accf[:n], accf[NPAD:NPAD + n]

    fo3 = W3.shape[1]
    W3p = jnp.pad(W3, ((0, 0), (0, 16 - fo3)))
    b3p = jnp.pad(b3, (0, 16 - fo3))

    a0, a1 = agg(_mm1(deg2, x, W1))
    a0, a1 = agg(_mid(deg2, a0, a1, b1.reshape(1, -1), W2))
    a0, a1 = agg(_mid(deg2, a0, a1, b2.reshape(1, -1), W3p))
    return _fin(deg2, a0, a1, b3p.reshape(1, -1))[:, :fo3]


# speedup vs baseline: 1.0590x; 1.0590x over previous
import jax
import jax.numpy as jnp
from jax import lax
from jax.experimental import pallas as pl
from jax.experimental.pallas import tpu as pltpu
from jax.experimental.pallas import tpu_sc as plsc

N_NODES = 10000
HROW = 640
NPAD = HROW * 16
NC, NS = 2, 16
NW = NC * NS
IDXB = 128
NB = 84
PER_TILE = NB * IDXB
E_PAD = NW * PER_TILE
NBATCH = E_PAD // IDXB
NB0 = 84
NB1 = (NBATCH - NS * NB0) // NS
BM = 2000


_SC_PARAMS = pltpu.CompilerParams(needs_layout_passes=False,
                                  use_tc_tiling_on_sc=False)


def _deg_body(col_hbm, idx_hbm, zeros_hbm, deg_hbm, colbuf, hist_v, idxv, shist):
    c = lax.axis_index("c")
    s = lax.axis_index("s")
    tid = c * NS + s
    pltpu.sync_copy(zeros_hbm, hist_v)
    pltpu.sync_copy(idx_hbm, idxv)
    pltpu.sync_copy(col_hbm.at[tid], colbuf)

    @pl.when(s == 0)
    def _():
        pltpu.sync_copy(zeros_hbm, shist)

    plsc.subcore_barrier()

    def outer(j, carry):
        def inner(l, carry):
            idx = colbuf[j, pl.ds(l * 16, 16)]
            cnt, last = plsc.scan_count(idx)
            hi = lax.shift_right_logical(idx, 4)
            lo = lax.bitwise_and(idx, 15)
            plsc.addupdate_scatter(hist_v, [hi, lo], cnt.astype(jnp.float32),
                                   mask=last)
            return carry
        return lax.fori_loop(0, IDXB // 16, inner, carry)

    lax.fori_loop(0, NB, outer, 0)

    for j in range(HROW // IDXB):
        pltpu.sync_copy(hist_v.at[pl.ds(j * IDXB, IDXB)],
                        shist.at[idxv.at[j]], add=True)
    plsc.subcore_barrier()

    @pl.when(s == 0)
    def _():
        pltpu.sync_copy(shist, deg_hbm.at[pl.ds(c * HROW, HROW)])


_deg_call = pl.kernel(
    _deg_body,
    out_type=jax.ShapeDtypeStruct((NC * HROW, 16), jnp.float32),
    mesh=plsc.VectorSubcoreMesh(core_axis_name="c", subcore_axis_name="s"),
    compiler_params=_SC_PARAMS,
    scratch_types=[
        pltpu.VMEM((NB, IDXB), jnp.int32),
        pltpu.VMEM((HROW, 16), jnp.float32),
        pltpu.VMEM((HROW // IDXB, IDXB), jnp.int32),
        pltpu.VMEM_SHARED((HROW, 16), jnp.float32),
    ],
)


NRING = 4


def _mp_body(xs_hbm, row_hbm, col_hbm, zeros_hbm, acc_hbm,
             rbuf, cbuf, g0, g1, g2, g3, acc_s, gsem, ssem):
    c = lax.axis_index("c")
    s = lax.axis_index("s")
    bufs = (g0, g1, g2, g3)
    start = pl.multiple_of(jnp.where(c == 0, s * NB0, NS * NB0 + s * NB1), 8)
    nb = jnp.where(c == 0, NB0, NB1)
    pltpu.sync_copy(row_hbm.at[pl.ds(start, NB1)], rbuf)
    pltpu.sync_copy(col_hbm.at[pl.ds(start, NB1)], cbuf)
    rows_per_tile = NPAD // NS
    pltpu.sync_copy(zeros_hbm.at[pl.ds(s * rows_per_tile, rows_per_tile)],
                    acc_s.at[pl.ds(s * rows_per_tile, rows_per_tile)])
    plsc.subcore_barrier()

    def wait_scat(buf):
        pltpu.make_async_copy(buf, acc_s.at[cbuf.at[0]], ssem).wait()

    for b in range(NRING):
        pltpu.async_copy(xs_hbm.at[rbuf.at[b]], bufs[b], gsem)

    def body(q, carry):
        j = q * NRING
        for b in range(NRING):
            pltpu.make_async_copy(xs_hbm.at[rbuf.at[j + b]], bufs[b],
                                  gsem).wait()
            pltpu.async_copy(bufs[b], acc_s.at[cbuf.at[j + b]], ssem,
                             add=True)
        for b in range(NRING):
            jn = j + NRING + b

            @pl.when(jn < nb)
            def _():
                wait_scat(bufs[b])
                pltpu.async_copy(xs_hbm.at[rbuf.at[jn]], bufs[b], gsem)

        return carry

    lax.fori_loop(0, nb // NRING, body, 0)
    for b in range(NRING):
        wait_scat(bufs[b])
    plsc.subcore_barrier()
    pltpu.sync_copy(acc_s.at[pl.ds(s * rows_per_tile, rows_per_tile)],
                    acc_hbm.at[pl.ds(c * NPAD + s * rows_per_tile,
                                     rows_per_tile)])


def _make_mp_call(feat):
    return pl.kernel(
        _mp_body,
        out_type=jax.ShapeDtypeStruct((NC * NPAD, feat), jnp.float32),
        mesh=plsc.VectorSubcoreMesh(core_axis_name="c", subcore_axis_name="s"),
        compiler_params=_SC_PARAMS,
        scratch_types=[
            pltpu.VMEM((NB1, IDXB), jnp.int32),
            pltpu.VMEM((NB1, IDXB), jnp.int32),
            pltpu.VMEM((IDXB, feat), jnp.float32),
            pltpu.VMEM((IDXB, feat), jnp.float32),
            pltpu.VMEM((IDXB, feat), jnp.float32),
            pltpu.VMEM((IDXB, feat), jnp.float32),
            pltpu.VMEM_SHARED((NPAD, feat), jnp.float32),
            pltpu.SemaphoreType.DMA,
            pltpu.SemaphoreType.DMA,
        ],
    )


_mp_calls = {feat: _make_mp_call(feat) for feat in (64, 32, 16)}


def _dis_block(deg):
    d = deg[:, 0] + deg[:, 1]
    return jnp.where(d > 0, lax.rsqrt(d), 0.0)[:, None]


def _mm1_body(deg_ref, x_ref, w_ref, o_ref):
    dis = _dis_block(deg_ref[...])
    o_ref[...] = dis * jnp.dot(x_ref[...], w_ref[...],
                               preferred_element_type=jnp.float32)


def _mid_body(deg_ref, a0_ref, a1_ref, b_ref, w_ref, o_ref):
    dis = _dis_block(deg_ref[...])
    h = jnp.maximum(dis * (a0_ref[...] + a1_ref[...]) + b_ref[...], 0.0)
    o_ref[...] = dis * jnp.dot(h, w_ref[...],
                               preferred_element_type=jnp.float32)


def _fin_body(deg_ref, a0_ref, a1_ref, b_ref, o_ref):
    dis = _dis_block(deg_ref[...])
    o_ref[...] = dis * (a0_ref[...] + a1_ref[...]) + b_ref[...]


def _row_spec(feat):
    return pl.BlockSpec((BM, feat), lambda m: (m, 0))


_DEG_SPEC = pl.BlockSpec((BM, 2), lambda m: (m, 0))
_GRID = (N_NODES // BM,)


def _mm1(deg, x, w):
    fin, fout = w.shape
    return pl.pallas_call(
        _mm1_body,
        grid=_GRID,
        in_specs=[_DEG_SPEC, _row_spec(fin),
                  pl.BlockSpec((fin, fout), lambda m: (0, 0))],
        out_specs=_row_spec(fout),
        out_shape=jax.ShapeDtypeStruct((N_NODES, fout), jnp.float32),
    )(deg, x, w)


def _mid(deg, a0, a1, b, w):
    fin, fout = w.shape
    return pl.pallas_call(
        _mid_body,
        grid=_GRID,
        in_specs=[_DEG_SPEC, _row_spec(fin), _row_spec(fin),
                  pl.BlockSpec((1, fin), lambda m: (0, 0)),
                  pl.BlockSpec((fin, fout), lambda m: (0, 0))],
        out_specs=_row_spec(fout),
        out_shape=jax.ShapeDtypeStruct((N_NODES, fout), jnp.float32),
    )(deg, a0, a1, b, w)


def _fin(deg, a0, a1, b):
    feat = a0.shape[1]
    return pl.pallas_call(
        _fin_body,
        grid=_GRID,
        in_specs=[_DEG_SPEC, _row_spec(feat), _row_spec(feat),
                  pl.BlockSpec((1, feat), lambda m: (0, 0))],
        out_specs=_row_spec(feat),
        out_shape=jax.ShapeDtypeStruct((N_NODES, feat), jnp.float32),
    )(deg, a0, a1, b)


def kernel(x, edge_index, W1, b1, W2, b2, W3, b3):
    n = x.shape[0]
    loop = jnp.arange(n, dtype=jnp.int32)
    row = jnp.concatenate([edge_index[0], loop])
    col = jnp.concatenate([edge_index[1], loop])
    npadding = E_PAD - row.shape[0]
    row_f = jnp.concatenate([row, jnp.zeros((npadding,), jnp.int32)])
    col_f = jnp.concatenate([col, jnp.full((npadding,), n, jnp.int32)])
    col_p3 = col_f.reshape(NW, NB, IDXB)
    row_p = row_f.reshape(NBATCH, IDXB)
    col_p = col_f.reshape(NBATCH, IDXB)
    idx_rows = jnp.arange(HROW, dtype=jnp.int32).reshape(HROW // IDXB, IDXB)
    zeros_deg = jnp.zeros((HROW, 16), jnp.float32)

    deg2 = _deg_call(col_p3, idx_rows, zeros_deg)
    deg2 = deg2.reshape(NC, NPAD)[:, :n].T

    def agg(xs):
        feat = xs.shape[1]
        accf = _mp_calls[feat](xs, row_p, col_p,
                               jnp.zeros((NPAD, feat), jnp.float32))
        return accf[:n], accf[NPAD:NPAD + n]

    fo3 = W3.shape[1]
    W3p = jnp.pad(W3, ((0, 0), (0, 16 - fo3)))
    b3p = jnp.pad(b3, (0, 16 - fo3))

    a0, a1 = agg(_mm1(deg2, x, W1))
    a0, a1 = agg(_mid(deg2, a0, a1, b1.reshape(1, -1), W2))
    a0, a1 = agg(_mid(deg2, a0, a1, b2.reshape(1, -1), W3p))
    return _fin(deg2, a0, a1, b3p.reshape(1, -1))[:, :fo3]
